# baseline (speedup 1.0000x reference)
import functools

import jax
import jax.numpy as jnp
from jax import lax
from jax.experimental import pallas as pl
from jax.experimental.pallas import tpu as pltpu
from jax.experimental.pallas import tpu_sc as plsc

VOCAB = 100000
EMBED_DIM = 64
BATCH = 4096
SEQ = 200

NUM_WORKERS = 32
ROWS_PER_W = BATCH // NUM_WORKERS
BLOCKS_PER_W = ROWS_PER_W // 16

_RB = 51200
_NB = -(-VOCAB // _RB)
_INV_SEQ = 1.0 / SEQ


def _score_body(t_ref, w_ref, b_ref, o_ref):
    t = t_ref[...]
    w = w_ref[...] * _INV_SEQ
    s = jax.lax.dot_general(w, t, (((1,), (0,)), ((), ())),
                            preferred_element_type=jnp.float32)
    o_ref[...] = (s + b_ref[0, 0] * _INV_SEQ).reshape(1, 1, _RB)


_score = pl.pallas_call(
    _score_body,
    grid=(_NB,),
    in_specs=[
        pl.BlockSpec((EMBED_DIM, _RB), lambda i: (0, i)),
        pl.BlockSpec((1, EMBED_DIM), lambda i: (0, 0)),
        pl.BlockSpec(memory_space=pltpu.SMEM),
    ],
    out_specs=pl.BlockSpec((1, 1, _RB), lambda i: (i, 0, 0)),
    out_shape=jax.ShapeDtypeStruct((_NB, 1, _RB), jnp.float32),
)


_NCHUNK = 2
_CHUNK = VOCAB // _NCHUNK


def _pool_body(s_hbm, xt_hbm, out_hbm, s_v, x_v, o_v, sem):
    wid = lax.axis_index("s") * 2 + lax.axis_index("c")
    cps = []
    for k in range(_NCHUNK):
        start = lax.rem(wid + k, _NCHUNK) * _CHUNK
        cps.append(pltpu.async_copy(s_hbm.at[pl.ds(start, _CHUNK)],
                                    s_v.at[pl.ds(start, _CHUNK)], sem))
    pltpu.sync_copy(xt_hbm.at[:, pl.ds(wid * ROWS_PER_W, ROWS_PER_W)],
                    x_v)
    for cp in cps:
        cp.wait()

    def body(l, accs):
        new = []
        for j in range(BLOCKS_PER_W):
            xv = x_v[l, pl.ds(j * 16, 16)]
            new.append(accs[j] + plsc.load_gather(s_v, [xv]))
        return tuple(new)

    zero = jnp.zeros((16,), jnp.float32)
    accs = lax.fori_loop(0, SEQ, body, (zero,) * BLOCKS_PER_W)
    for j in range(BLOCKS_PER_W):
        o_v[pl.ds(j * 16, 16)] = accs[j]
    pltpu.sync_copy(o_v, out_hbm.at[pl.ds(wid * ROWS_PER_W, ROWS_PER_W)])


_pool = functools.partial(
    pl.kernel,
    mesh=plsc.VectorSubcoreMesh(core_axis_name="c", subcore_axis_name="s"),
    compiler_params=pltpu.CompilerParams(needs_layout_passes=False),
    out_type=jax.ShapeDtypeStruct((BATCH,), jnp.float32),
    scratch_types=[
        pltpu.VMEM((VOCAB,), jnp.float32),
        pltpu.VMEM((SEQ, ROWS_PER_W), jnp.int32),
        pltpu.VMEM((ROWS_PER_W,), jnp.float32),
        pltpu.SemaphoreType.DMA,
    ],
)(_pool_body)


def kernel(x, table, W, b):
    s = _score(table.T, W.astype(jnp.float32),
               b.astype(jnp.float32).reshape(1, 1)).reshape(_NB * _RB)
    return _pool(s, x.astype(jnp.int32).T)


# speedup vs baseline: 1.0225x; 1.0225x over previous
import functools

import jax
import jax.numpy as jnp
from jax import lax
from jax.experimental import pallas as pl
from jax.experimental.pallas import tpu as pltpu
from jax.experimental.pallas import tpu_sc as plsc

VOCAB = 100000
EMBED_DIM = 64
BATCH = 4096
SEQ = 200

NUM_WORKERS = 32
ROWS_PER_W = BATCH // NUM_WORKERS
BLOCKS_PER_W = ROWS_PER_W // 16

_RB = 51200
_NB = -(-VOCAB // _RB)
_INV_SEQ = 1.0 / SEQ


def _score_body(t_ref, w_ref, b_ref, o_ref):
    t = t_ref[...]
    w = w_ref[...] * _INV_SEQ
    s = jax.lax.dot_general(w, t, (((1,), (0,)), ((), ())),
                            preferred_element_type=jnp.float32)
    o_ref[...] = (s + b_ref[0, 0] * _INV_SEQ).reshape(1, 1, _RB)


_score = pl.pallas_call(
    _score_body,
    grid=(_NB,),
    in_specs=[
        pl.BlockSpec((EMBED_DIM, _RB), lambda i: (0, i)),
        pl.BlockSpec((1, EMBED_DIM), lambda i: (0, 0)),
        pl.BlockSpec(memory_space=pltpu.SMEM),
    ],
    out_specs=pl.BlockSpec((1, 1, _RB), lambda i: (i, 0, 0)),
    out_shape=jax.ShapeDtypeStruct((_NB, 1, _RB), jnp.float32),
)


_NCHUNK = 5
_CHUNK = VOCAB // _NCHUNK


def _pool_body(s_hbm, xt_hbm, out_hbm, s_v, x_v, o_v, sem):
    wid = lax.axis_index("s") * 2 + lax.axis_index("c")
    cps = []
    for k in range(_NCHUNK):
        start = lax.rem(wid + k, _NCHUNK) * _CHUNK
        cps.append(pltpu.async_copy(s_hbm.at[pl.ds(start, _CHUNK)],
                                    s_v.at[pl.ds(start, _CHUNK)], sem))
    pltpu.sync_copy(xt_hbm.at[:, pl.ds(wid * ROWS_PER_W, ROWS_PER_W)],
                    x_v)
    for cp in cps:
        cp.wait()

    def body(l, accs):
        new = []
        for j in range(BLOCKS_PER_W):
            xv = x_v[l, pl.ds(j * 16, 16)]
            new.append(accs[j] + plsc.load_gather(s_v, [xv]))
        return tuple(new)

    zero = jnp.zeros((16,), jnp.float32)
    accs = lax.fori_loop(0, SEQ, body, (zero,) * BLOCKS_PER_W)
    for j in range(BLOCKS_PER_W):
        o_v[pl.ds(j * 16, 16)] = accs[j]
    pltpu.sync_copy(o_v, out_hbm.at[pl.ds(wid * ROWS_PER_W, ROWS_PER_W)])


_pool = functools.partial(
    pl.kernel,
    mesh=plsc.VectorSubcoreMesh(core_axis_name="c", subcore_axis_name="s"),
    compiler_params=pltpu.CompilerParams(needs_layout_passes=False),
    out_type=jax.ShapeDtypeStruct((BATCH,), jnp.float32),
    scratch_types=[
        pltpu.VMEM((VOCAB,), jnp.float32),
        pltpu.VMEM((SEQ, ROWS_PER_W), jnp.int32),
        pltpu.VMEM((ROWS_PER_W,), jnp.float32),
        pltpu.SemaphoreType.DMA,
    ],
)(_pool_body)


def kernel(x, table, W, b):
    s = _score(table.T, W.astype(jnp.float32),
               b.astype(jnp.float32).reshape(1, 1)).reshape(_NB * _RB)
    return _pool(s, x.astype(jnp.int32).T)
